# baseline (speedup 1.0000x reference)
import jax
import jax.numpy as jnp
from jax import lax
from jax.experimental import pallas as pl
from jax.experimental.pallas import tpu as pltpu
from jax.experimental.pallas import tpu_sc as plsc

N_TOTAL = 32768
N_GROUPS = 16
CHUNK = N_TOTAL // 32
WSIZE = 5120


def _body(scores_hbm, nbest_hbm, out_hbm, nb_v, window, outbuf, redbuf):
    cid = lax.axis_index("c")
    sid = lax.axis_index("s")
    wid = sid * 2 + cid
    lane = lax.iota(jnp.int32, 16)

    pltpu.sync_copy(nbest_hbm, nb_v)
    nb = nb_v[...]
    run = jnp.int32(0)
    starts_s, ends_s = [], []
    for g in range(N_GROUPS):
        starts_s.append(run)
        run = run + nb[g]
        ends_s.append(run)

    c0 = wid * CHUNK
    c1 = c0 + CHUNK
    ws = pl.multiple_of(
        jnp.minimum(jnp.maximum(c0 - 2048, 0), N_TOTAL - WSIZE), CHUNK)
    pltpu.sync_copy(scores_hbm.at[pl.ds(ws, WSIZE)], window)

    coff = c0 - ws
    for j in range(CHUNK // 16):
        outbuf[pl.ds(j * 16, 16)] = window[pl.ds(coff + j * 16, 16)]

    for g in range(N_GROUPS):
        s_g, e_g = starts_s[g], ends_s[g]

        @pl.when((s_g < c1) & (e_g > c0))
        def _segment(s_g=s_g, e_g=e_g):
            ilo = s_g >> 4
            ihi = (e_g + 15) >> 4
            iflo = (s_g + 15) >> 4
            ifhi = jnp.maximum(e_g >> 4, iflo)

            def msum(i, s):
                gpos = i * 16 + lane
                v = window[pl.ds(i * 16 - ws, 16)]
                m = (gpos >= s_g) & (gpos < e_g)
                return s + jnp.where(m, jnp.exp(v), 0.0)

            s = lax.fori_loop(ilo, jnp.minimum(iflo, ihi), msum,
                              jnp.zeros((16,), jnp.float32))
            n_int = ifhi - iflo

            def usum4(k, s):
                b = (iflo + k * 4) * 16 - ws
                for t in range(4):
                    s = s + jnp.exp(window[pl.ds(b + t * 16, 16)])
                return s

            s = lax.fori_loop(0, n_int >> 2, usum4, s)

            def usum(i, s):
                return s + jnp.exp(window[pl.ds(i * 16 - ws, 16)])

            s = lax.fori_loop(iflo + (n_int & -4), ifhi, usum, s)
            s = lax.fori_loop(ifhi, ihi, msum, s)

            for k in (1, 2, 4, 8):
                redbuf[...] = s
                s = s + plsc.load_gather(redbuf, [lane ^ k])
            inv = 1.0 / s

            cs = jnp.maximum(s_g, c0)
            ce = jnp.minimum(e_g, c1)
            jlo = cs >> 4
            jhi = (ce + 15) >> 4
            jflo = (cs + 15) >> 4
            jfhi = jnp.maximum(ce >> 4, jflo)

            def mnrm(i, carry):
                gpos = i * 16 + lane
                v = window[pl.ds(i * 16 - ws, 16)]
                m = (gpos >= s_g) & (gpos < e_g)
                o = outbuf[pl.ds(i * 16 - c0, 16)]
                outbuf[pl.ds(i * 16 - c0, 16)] = jnp.where(
                    m, jnp.exp(v) * inv, o)
                return carry

            def unrm(i, carry):
                outbuf[pl.ds(i * 16 - c0, 16)] = (
                    jnp.exp(window[pl.ds(i * 16 - ws, 16)]) * inv)
                return carry

            lax.fori_loop(jlo, jnp.minimum(jflo, jhi), mnrm, jnp.int32(0))
            lax.fori_loop(jflo, jfhi, unrm, jnp.int32(0))
            lax.fori_loop(jfhi, jhi, mnrm, jnp.int32(0))

    pltpu.sync_copy(outbuf, out_hbm.at[pl.ds(c0, CHUNK)])


@jax.jit
def kernel(scores, nBestIndex):
    mesh = plsc.VectorSubcoreMesh(core_axis_name="c", subcore_axis_name="s")
    f = pl.kernel(
        _body,
        out_type=jax.ShapeDtypeStruct((N_TOTAL,), jnp.float32),
        mesh=mesh,
        compiler_params=pltpu.CompilerParams(
            needs_layout_passes=False,
            skip_device_barrier=True,
            disable_bounds_checks=True,
            disable_semaphore_checks=True,
        ),
        scratch_types=[
            pltpu.VMEM((N_GROUPS,), jnp.int32),
            pltpu.VMEM((WSIZE,), jnp.float32),
            pltpu.VMEM((CHUNK,), jnp.float32),
            pltpu.VMEM((16,), jnp.float32),
        ],
    )
    return f(scores, nBestIndex)


# speedup vs baseline: 1.5080x; 1.5080x over previous
import jax
import jax.numpy as jnp
from jax import lax
from jax.experimental import pallas as pl
from jax.experimental.pallas import tpu as pltpu
from jax.experimental.pallas import tpu_sc as plsc

N_TOTAL = 32768
N_GROUPS = 16
CHUNK = N_TOTAL // 32
WSIZE = 5120


def _body(scores_hbm, nbest_hbm, out_hbm, nb_v, window, outbuf, sbuf, ebuf,
          redbuf, ibuf):
    cid = lax.axis_index("c")
    sid = lax.axis_index("s")
    wid = sid * 2 + cid
    lane = lax.iota(jnp.int32, 16)

    pltpu.sync_copy(nbest_hbm, nb_v)
    nb = nb_v[...]
    x = nb
    for k in (1, 2, 4, 8):
        ibuf[...] = x
        y = plsc.load_gather(ibuf, [jnp.maximum(lane - k, 0)])
        x = x + jnp.where(lane >= k, y, 0)
    ends_v = x
    starts_v = ends_v - nb
    sbuf[...] = starts_v
    ebuf[...] = ends_v

    c0 = wid * CHUNK
    c1 = c0 + CHUNK
    ws = pl.multiple_of(
        jnp.minimum(jnp.maximum(c0 - 2048, 0), N_TOTAL - WSIZE), CHUNK)
    pltpu.sync_copy(scores_hbm.at[pl.ds(ws, WSIZE)], window)

    coff = c0 - ws
    for j in range(CHUNK // 16):
        outbuf[pl.ds(j * 16, 16)] = window[pl.ds(coff + j * 16, 16)]

    def seg_loop(g, carry):
        gv = jnp.zeros((16,), jnp.int32) + g
        s_g = plsc.load_gather(sbuf, [gv])[0]
        e_g = plsc.load_gather(ebuf, [gv])[0]

        @pl.when((s_g < c1) & (e_g > c0))
        def _segment():
            ilo = s_g >> 4
            ihi = (e_g + 15) >> 4
            iflo = (s_g + 15) >> 4
            ifhi = jnp.maximum(e_g >> 4, iflo)

            def msum(i, s):
                gpos = i * 16 + lane
                v = window[pl.ds(i * 16 - ws, 16)]
                m = (gpos >= s_g) & (gpos < e_g)
                return s + jnp.where(m, jnp.exp(v), 0.0)

            s = lax.fori_loop(ilo, jnp.minimum(iflo, ihi), msum,
                              jnp.zeros((16,), jnp.float32))
            n_int = ifhi - iflo

            def usum4(k, s):
                b = (iflo + k * 4) * 16 - ws
                e0 = jnp.exp(window[pl.ds(b, 16)])
                e1 = jnp.exp(window[pl.ds(b + 16, 16)])
                e2 = jnp.exp(window[pl.ds(b + 32, 16)])
                e3 = jnp.exp(window[pl.ds(b + 48, 16)])
                return s + ((e0 + e1) + (e2 + e3))

            s = lax.fori_loop(0, n_int >> 2, usum4, s)

            def usum(i, s):
                return s + jnp.exp(window[pl.ds(i * 16 - ws, 16)])

            s = lax.fori_loop(iflo + (n_int & -4), ifhi, usum, s)
            s = lax.fori_loop(ifhi, ihi, msum, s)

            for k in (1, 2, 4, 8):
                redbuf[...] = s
                s = s + plsc.load_gather(redbuf, [lane ^ k])
            inv = 1.0 / s

            cs = jnp.maximum(s_g, c0)
            ce = jnp.minimum(e_g, c1)
            jlo = cs >> 4
            jhi = (ce + 15) >> 4
            jflo = (cs + 15) >> 4
            jfhi = jnp.maximum(ce >> 4, jflo)

            def mnrm(i, carry2):
                gpos = i * 16 + lane
                v = window[pl.ds(i * 16 - ws, 16)]
                m = (gpos >= s_g) & (gpos < e_g)
                o = outbuf[pl.ds(i * 16 - c0, 16)]
                outbuf[pl.ds(i * 16 - c0, 16)] = jnp.where(
                    m, jnp.exp(v) * inv, o)
                return carry2

            def unrm(i, carry2):
                outbuf[pl.ds(i * 16 - c0, 16)] = (
                    jnp.exp(window[pl.ds(i * 16 - ws, 16)]) * inv)
                return carry2

            lax.fori_loop(jlo, jnp.minimum(jflo, jhi), mnrm, jnp.int32(0))
            lax.fori_loop(jflo, jfhi, unrm, jnp.int32(0))
            lax.fori_loop(jfhi, jhi, mnrm, jnp.int32(0))

        return carry

    lax.fori_loop(0, N_GROUPS, seg_loop, jnp.int32(0))

    pltpu.sync_copy(outbuf, out_hbm.at[pl.ds(c0, CHUNK)])


@jax.jit
def kernel(scores, nBestIndex):
    mesh = plsc.VectorSubcoreMesh(core_axis_name="c", subcore_axis_name="s")
    f = pl.kernel(
        _body,
        out_type=jax.ShapeDtypeStruct((N_TOTAL,), jnp.float32),
        mesh=mesh,
        compiler_params=pltpu.CompilerParams(
            needs_layout_passes=False,
            skip_device_barrier=True,
            disable_bounds_checks=True,
            disable_semaphore_checks=True,
        ),
        scratch_types=[
            pltpu.VMEM((N_GROUPS,), jnp.int32),
            pltpu.VMEM((WSIZE,), jnp.float32),
            pltpu.VMEM((CHUNK,), jnp.float32),
            pltpu.VMEM((N_GROUPS,), jnp.int32),
            pltpu.VMEM((N_GROUPS,), jnp.int32),
            pltpu.VMEM((16,), jnp.float32),
            pltpu.VMEM((16,), jnp.int32),
        ],
    )
    return f(scores, nBestIndex)
